# baseline (device time: 1195234 ns/iter reference)
import jax
import jax.numpy as jnp
from jax import lax
from jax.experimental import pallas as pl
from jax.experimental.pallas import tpu as pltpu

N_DEV = 8
TILE = 512
HOPS_R = 4
HOPS_L = 3


def kernel(A, B):
    a16 = A.astype(jnp.bfloat16)
    b16 = B.astype(jnp.bfloat16)
    m_per, k = a16.shape
    n = b16.shape[1]
    n_tiles = m_per // TILE

    def body(a_ref, b_ref, out_ref, ag_ref, a_tile, c_tile,
             a_sems, c_sems, send_r, recv_r, send_l, recv_l):
        my = lax.axis_index("i")
        left = lax.rem(my + N_DEV - 1, N_DEV)
        right = lax.rem(my + 1, N_DEV)

        def rows(blk):
            return pl.ds(lax.rem(blk + N_DEV, N_DEV) * m_per, m_per)

        barrier_sem = pltpu.get_barrier_semaphore()
        for nbr in (left, right):
            pl.semaphore_signal(
                barrier_sem, inc=1,
                device_id=(nbr,), device_id_type=pl.DeviceIdType.MESH,
            )
        pl.semaphore_wait(barrier_sem, 2)

        def make_hop(h, to_right):
            blk = my - h if to_right else my + h
            src = a_ref if h == 0 else ag_ref.at[rows(blk)]
            return pltpu.make_async_remote_copy(
                src_ref=src,
                dst_ref=ag_ref.at[rows(blk)],
                send_sem=(send_r if to_right else send_l).at[h],
                recv_sem=(recv_r if to_right else recv_l).at[h],
                device_id=(right if to_right else left,),
                device_id_type=pl.DeviceIdType.MESH,
            )

        block_count = [0]

        def gemm_block(src_ref, row0, out0):
            first = block_count[0] == 0
            block_count[0] += 1

            def a_load(t, s):
                return pltpu.make_async_copy(
                    src_ref.at[pl.ds(row0 + t * TILE, TILE)],
                    a_tile.at[s], a_sems.at[s],
                )

            def c_store(t, s):
                return pltpu.make_async_copy(
                    c_tile.at[s],
                    out_ref.at[pl.ds(out0 + t * TILE, TILE)],
                    c_sems.at[s],
                )

            a_load(0, 0).start()

            def tile_step(t, carry):
                s = lax.rem(t, 2)
                a_load(t, s).wait()

                @pl.when(t + 1 < n_tiles)
                def _():
                    a_load(t + 1, 1 - s).start()

                if first:
                    @pl.when(t >= 2)
                    def _():
                        c_store(t, s).wait()
                else:
                    c_store(t, s).wait()

                c_tile[s] = jnp.dot(
                    a_tile[s], b_ref[...],
                    preferred_element_type=jnp.float32,
                )
                c_store(t, s).start()
                return carry

            lax.fori_loop(0, n_tiles, tile_step, 0)

        hops_r = [make_hop(0, True)]
        hops_l = [make_hop(0, False)]
        hops_r[0].start()
        hops_l[0].start()
        gemm_block(a_ref, 0, my * m_per)

        for h in range(HOPS_R):
            hops_r[h].wait_recv()
            if h < HOPS_L:
                hops_l[h].wait_recv()
            if h + 1 < HOPS_R:
                nxt = make_hop(h + 1, True)
                nxt.start()
                hops_r.append(nxt)
            if h + 1 < HOPS_L:
                nxt = make_hop(h + 1, False)
                nxt.start()
                hops_l.append(nxt)
            blk_r = lax.rem(my - 1 - h + N_DEV, N_DEV)
            gemm_block(ag_ref, blk_r * m_per, blk_r * m_per)
            if h < HOPS_L:
                blk_l = lax.rem(my + 1 + h, N_DEV)
                gemm_block(ag_ref, blk_l * m_per, blk_l * m_per)

        for s in (0, 1):
            pltpu.make_async_copy(
                c_tile.at[s], out_ref.at[pl.ds(0, TILE)], c_sems.at[s]
            ).wait()
        for rdma in hops_r + hops_l:
            rdma.wait_send()

    out, _ = pl.pallas_call(
        body,
        out_shape=(
            jax.ShapeDtypeStruct((N_DEV * m_per, n), jnp.float32),
            jax.ShapeDtypeStruct((N_DEV * m_per, k), jnp.bfloat16),
        ),
        in_specs=[
            pl.BlockSpec(memory_space=pl.ANY),
            pl.BlockSpec(memory_space=pltpu.VMEM),
        ],
        out_specs=(
            pl.BlockSpec(memory_space=pl.ANY),
            pl.BlockSpec(memory_space=pl.ANY),
        ),
        scratch_shapes=[
            pltpu.VMEM((2, TILE, k), jnp.bfloat16),
            pltpu.VMEM((2, TILE, n), jnp.float32),
            pltpu.SemaphoreType.DMA((2,)),
            pltpu.SemaphoreType.DMA((2,)),
            pltpu.SemaphoreType.DMA((HOPS_R,)),
            pltpu.SemaphoreType.DMA((HOPS_R,)),
            pltpu.SemaphoreType.DMA((HOPS_L,)),
            pltpu.SemaphoreType.DMA((HOPS_L,)),
        ],
        compiler_params=pltpu.CompilerParams(
            collective_id=0,
            vmem_limit_bytes=62 * 1024 * 1024,
        ),
    )(a16, b16)
    return out
